# baseline (speedup 1.0000x reference)
import functools

import jax
import jax.numpy as jnp
from jax import lax
from jax.experimental import pallas as pl
from jax.experimental.pallas import tpu as pltpu
from jax.experimental.pallas import tpu_sc as plsc

B = 16384
F = 26
V = 100000
D = 16

NC = 2
NS = 16
NW = NC * NS
NPAIR = F * D
PAIRS_PER_W = NPAIR // NW
VH0 = 50048
VH1 = V - VH0
CHB = 4096
NCHUNK = B // CHB


def _sc_gather_t(idx_t, tab_t):
    mesh = plsc.VectorSubcoreMesh(core_axis_name="c", subcore_axis_name="s")

    @functools.partial(
        pl.kernel,
        out_type=jax.ShapeDtypeStruct((NPAIR, B), jnp.float32),
        mesh=mesh,
        scratch_types=[
            pltpu.VMEM((VH0,), jnp.float32),
            pltpu.VMEM((VH1,), jnp.float32),
            pltpu.VMEM((B,), jnp.int32),
            pltpu.VMEM((CHB,), jnp.float32),
            pltpu.VMEM((CHB,), jnp.float32),
            pltpu.SemaphoreType.DMA,
            pltpu.SemaphoreType.DMA,
            pltpu.SemaphoreType.DMA,
            pltpu.SemaphoreType.DMA,
        ],
        compiler_params=pltpu.CompilerParams(needs_layout_passes=False),
    )
    def k(idx_hbm, tab_hbm, out_hbm, slab_a, slab_b, idx_v, out_v0, out_v1,
          sa, sb, so0, so1):
        wid = lax.axis_index("s") * NC + lax.axis_index("c")
        osems = (so0, so1)
        obufs = (out_v0, out_v1)

        def slab_copies(pair):
            f = pair // D
            d = pair % D
            cpa = pltpu.async_copy(
                tab_hbm.at[f, d, pl.ds(0, VH0)], slab_a, sa)
            cpb = pltpu.async_copy(
                tab_hbm.at[f, d, pl.ds(VH0, VH1)], slab_b, sb)
            return cpa, cpb

        p0 = wid * PAIRS_PER_W
        cpa, cpb = slab_copies(p0)
        out_cp = [None, None]

        for i in range(PAIRS_PER_W):
            p = p0 + i
            f = p // D
            if i == 0:
                pltpu.sync_copy(idx_hbm.at[f], idx_v)
            else:
                @pl.when(f != (p - 1) // D)
                def _():
                    pltpu.sync_copy(idx_hbm.at[f], idx_v)

            cpa.wait()
            next_cp = [None, None]

            for cb in range(NCHUNK):
                buf = cb % 2
                ob = obufs[buf]
                if out_cp[buf] is not None:
                    out_cp[buf].wait()

                @plsc.parallel_loop(0, CHB, step=16, unroll=8)
                def _(j):
                    vidx = idx_v[pl.ds(cb * CHB + j, 16)]
                    ob[pl.ds(j, 16)] = plsc.load_gather(
                        slab_a, [jnp.minimum(vidx, VH0 - 1)])

                if cb == 0:
                    cpb.wait()
                if cb == NCHUNK - 1 and i + 1 < PAIRS_PER_W:
                    next_cp[0] = pltpu.async_copy(
                        tab_hbm.at[(p + 1) // D, (p + 1) % D, pl.ds(0, VH0)],
                        slab_a, sa)

                @plsc.parallel_loop(0, CHB, step=16, unroll=8)
                def _(j):
                    vidx = idx_v[pl.ds(cb * CHB + j, 16)]
                    mask = vidx >= VH0
                    hidx = jnp.maximum(vidx - VH0, 0)
                    vals = plsc.load_gather(slab_b, [hidx], mask=mask)
                    pos = lax.iota(jnp.int32, 16) + j
                    plsc.store_scatter(ob, [pos], vals, mask=mask)

                out_cp[buf] = pltpu.async_copy(
                    ob, out_hbm.at[p, pl.ds(cb * CHB, CHB)],
                    osems[buf])

            if i + 1 < PAIRS_PER_W:
                next_cp[1] = pltpu.async_copy(
                    tab_hbm.at[(p + 1) // D, (p + 1) % D, pl.ds(VH0, VH1)],
                    slab_b, sb)
                cpa, cpb = next_cp

        out_cp[0].wait()
        out_cp[1].wait()

    return k(idx_t, tab_t)


def kernel(indices, tables):
    idx_t = indices.T
    tab_t = tables.transpose(0, 2, 1)
    out_t = _sc_gather_t(idx_t, tab_t)
    return out_t.T


# speedup vs baseline: 1.0179x; 1.0179x over previous
import functools

import jax
import jax.numpy as jnp
from jax import lax
from jax.experimental import pallas as pl
from jax.experimental.pallas import tpu as pltpu
from jax.experimental.pallas import tpu_sc as plsc

B = 16384
F = 26
V = 100000
D = 16

NC = 2
NS = 16
NW = NC * NS
NPAIR = F * D
PAIRS_PER_W = NPAIR // NW
VH0 = 50048
VH1 = V - VH0
CHB = 4096
NCHUNK = B // CHB


def _sc_gather_t(idx_t, tab_t):
    mesh = plsc.VectorSubcoreMesh(core_axis_name="c", subcore_axis_name="s")

    @functools.partial(
        pl.kernel,
        out_type=jax.ShapeDtypeStruct((NPAIR, B), jnp.float32),
        mesh=mesh,
        scratch_types=[
            pltpu.VMEM((VH0,), jnp.float32),
            pltpu.VMEM((VH1,), jnp.float32),
            pltpu.VMEM((B,), jnp.int32),
            pltpu.VMEM((CHB,), jnp.float32),
            pltpu.VMEM((CHB,), jnp.float32),
            pltpu.SemaphoreType.DMA,
            pltpu.SemaphoreType.DMA,
            pltpu.SemaphoreType.DMA,
            pltpu.SemaphoreType.DMA,
        ],
        compiler_params=pltpu.CompilerParams(needs_layout_passes=False),
    )
    def k(idx_hbm, tab_hbm, out_hbm, slab_a, slab_b, idx_v, out_v0, out_v1,
          sa, sb, so0, so1):
        wid = lax.axis_index("s") * NC + lax.axis_index("c")
        osems = (so0, so1)
        obufs = (out_v0, out_v1)

        def slab_copies(pair):
            f = pair // D
            d = pair % D
            cpa = pltpu.async_copy(
                tab_hbm.at[f, d, pl.ds(0, VH0)], slab_a, sa)
            cpb = pltpu.async_copy(
                tab_hbm.at[f, d, pl.ds(VH0, VH1)], slab_b, sb)
            return cpa, cpb

        p0 = wid * PAIRS_PER_W
        cpa, cpb = slab_copies(p0)
        out_cp = [None, None]

        for i in range(PAIRS_PER_W):
            p = p0 + i
            f = p // D
            if i == 0:
                pltpu.sync_copy(idx_hbm.at[f], idx_v)
            else:
                @pl.when(f != (p - 1) // D)
                def _():
                    pltpu.sync_copy(idx_hbm.at[f], idx_v)

            cpa.wait()
            next_cp = [None, None]

            def pass_a(cb):
                ob = obufs[cb % 2]
                if out_cp[cb % 2] is not None:
                    out_cp[cb % 2].wait()
                    out_cp[cb % 2] = None

                @plsc.parallel_loop(0, CHB, step=16, unroll=8)
                def _(j):
                    vidx = idx_v[pl.ds(cb * CHB + j, 16)]
                    ob[pl.ds(j, 16)] = plsc.load_gather(
                        slab_a, [jnp.minimum(vidx, VH0 - 1)])

            def pass_b(cb):
                ob = obufs[cb % 2]

                @plsc.parallel_loop(0, CHB, step=16, unroll=8)
                def _(j):
                    vidx = idx_v[pl.ds(cb * CHB + j, 16)]
                    mask = vidx >= VH0
                    hidx = jnp.maximum(vidx - VH0, 0)
                    vals = plsc.load_gather(slab_b, [hidx], mask=mask)
                    pos = lax.iota(jnp.int32, 16) + j
                    plsc.store_scatter(ob, [pos], vals, mask=mask)

                out_cp[cb % 2] = pltpu.async_copy(
                    obufs[cb % 2], out_hbm.at[p, pl.ds(cb * CHB, CHB)],
                    osems[cb % 2])

            pass_a(0)
            pass_a(1)
            cpb.wait()
            for cb in range(NCHUNK):
                pass_b(cb)
                if cb + 2 < NCHUNK:
                    pass_a(cb + 2)
                    if cb + 2 == NCHUNK - 1 and i + 1 < PAIRS_PER_W:
                        next_cp[0] = pltpu.async_copy(
                            tab_hbm.at[(p + 1) // D, (p + 1) % D,
                                       pl.ds(0, VH0)],
                            slab_a, sa)

            if i + 1 < PAIRS_PER_W:
                next_cp[1] = pltpu.async_copy(
                    tab_hbm.at[(p + 1) // D, (p + 1) % D, pl.ds(VH0, VH1)],
                    slab_b, sb)
                cpa, cpb = next_cp

        out_cp[0].wait()
        out_cp[1].wait()

    return k(idx_t, tab_t)


def kernel(indices, tables):
    idx_t = indices.T
    tab_t = tables.transpose(0, 2, 1)
    out_t = _sc_gather_t(idx_t, tab_t)
    return out_t.T
